# Initial kernel scaffold; baseline (speedup 1.0000x reference)
import jax
import jax.numpy as jnp
from jax.experimental import pallas as pl


def kernel(inputs, cluster_mean):
    raise NotImplementedError("write your pallas kernel here")


# speedup vs baseline: 1.5112x; 1.5112x over previous
import functools

import jax
import jax.numpy as jnp
from jax import lax
from jax.experimental import pallas as pl
from jax.experimental.pallas import tpu as pltpu
from jax.experimental.pallas import tpu_sc as plsc

D = 256
N = 8192
M = 16384
BM = 512
N_STEPS = M // BM

NC = 2
NS = 16
NW = NC * NS
B_PER_W = M // NW
CHUNK = 128
N_CHUNKS = B_PER_W // CHUNK


def _dist_argmin_body(s_ref, cm_ref, rown_ref, cn_ref, idx_ref, loss_ref):
    i = pl.program_id(0)

    @pl.when(i == 0)
    def _init():
        loss_ref[...] = jnp.zeros((1, 1), jnp.float32)

    mm = jnp.dot(s_ref[...].astype(jnp.bfloat16), cm_ref[...].astype(jnp.bfloat16),
                 preferred_element_type=jnp.float32)
    d = (rown_ref[...] - 2.0 * mm) + cn_ref[...]
    dmin = jnp.min(d, axis=1, keepdims=True)
    cols = lax.broadcasted_iota(jnp.int32, (BM, N), 1)
    idx = jnp.min(jnp.where(d == dmin, cols, jnp.int32(2**30)), axis=1)
    idx_ref[...] = idx.reshape(1, 1, BM)
    loss_ref[...] = loss_ref[...] + jnp.sum(dmin)

    @pl.when(i == N_STEPS - 1)
    def _finish():
        lv = loss_ref[0, 0] * (1.0 / (M * D))
        loss_ref[...] = jnp.full((1, 1), lv + 0.25 * lv, jnp.float32)


_dist_argmin = pl.pallas_call(
    _dist_argmin_body,
    grid=(N_STEPS,),
    in_specs=[
        pl.BlockSpec((BM, D), lambda i: (i, 0)),
        pl.BlockSpec((D, N), lambda i: (0, 0)),
        pl.BlockSpec((BM, 1), lambda i: (i, 0)),
        pl.BlockSpec((1, N), lambda i: (0, 0)),
    ],
    out_specs=[
        pl.BlockSpec((1, 1, BM), lambda i: (i, 0, 0)),
        pl.BlockSpec((1, 1), lambda i: (0, 0)),
    ],
    out_shape=[
        jax.ShapeDtypeStruct((N_STEPS, 1, BM), jnp.int32),
        jax.ShapeDtypeStruct((1, 1), jnp.float32),
    ],
)


@functools.cache
def _make_sc_gather():
    @functools.partial(
        pl.kernel,
        mesh=plsc.VectorSubcoreMesh(core_axis_name="c", subcore_axis_name="s"),
        out_type=jax.ShapeDtypeStruct((M, D), jnp.float32),
        scratch_types=[
            pltpu.VMEM((CHUNK,), jnp.int32),
            pltpu.VMEM((CHUNK, D), jnp.float32),
            pltpu.SemaphoreType.DMA,
        ],
    )
    def _sc_gather(table_hbm, idx_hbm, out_hbm, idx_v, rows_v, sem):
        wid = lax.axis_index("s") * NC + lax.axis_index("c")
        base = wid * B_PER_W
        for c in range(N_CHUNKS):
            off = base + c * CHUNK
            pltpu.sync_copy(idx_hbm.at[pl.ds(off, CHUNK)], idx_v)
            pltpu.async_copy(table_hbm.at[idx_v], rows_v, sem).wait()
            pltpu.sync_copy(rows_v, out_hbm.at[pl.ds(off, CHUNK)])

    return _sc_gather


def kernel(inputs, cluster_mean):
    B, H, W, _ = inputs.shape
    samples = inputs.reshape(M, D)
    rown = jnp.sum(jnp.power(samples, 2), axis=1, keepdims=True)
    cn = jnp.sum(jnp.power(cluster_mean, 2), axis=0, keepdims=True)
    idx_blocks, loss = _dist_argmin(samples, cluster_mean, rown, cn)
    idx = idx_blocks.reshape(M)
    quantize = _make_sc_gather()(cluster_mean.T, idx)
    return (
        quantize.reshape(B, H, W, D),
        idx.reshape(B, H, W),
        loss[0, 0],
    )
